# Initial kernel scaffold; baseline (speedup 1.0000x reference)
import jax
import jax.numpy as jnp
from jax.experimental import pallas as pl


def kernel(x, edge_index, W1_l, b1, W1_r, W2_l, b2, W2_r):
    raise NotImplementedError("write your pallas kernel here")


# speedup vs baseline: 6.9265x; 6.9265x over previous
import jax
import jax.numpy as jnp
from jax import lax
from jax.experimental import pallas as pl
from jax.experimental.pallas import tpu as pltpu
from jax.experimental.pallas import tpu_sc as plsc

_NC = 2
_NS = 16
_NW = _NC * _NS
_LANES = 16
_B = 80


def _make_edge_agg(n, e, d, with_deg):
  epw = e // _NW
  assert epw * _NW == e and epw % _B == 0
  nchunks = epw // _B
  rpt = n // _NS
  assert rpt * _NS == n and rpt % 8 == 0

  mesh = plsc.VectorSubcoreMesh(core_axis_name="c", subcore_axis_name="s")
  out_type = [jax.ShapeDtypeStruct((_NC, n, d), jnp.float32)]
  scratch = [
      pltpu.VMEM((_B,), jnp.int32),
      pltpu.VMEM((_B,), jnp.int32),
      pltpu.VMEM((_B, d), jnp.float32),
      pltpu.VMEM_SHARED((n, d), jnp.float32),
      pltpu.SemaphoreType.DMA,
  ]
  if with_deg:
    out_type.append(jax.ShapeDtypeStruct((_NC, n, _LANES), jnp.float32))
    scratch += [
        pltpu.VMEM((_B, _LANES), jnp.float32),
        pltpu.VMEM_SHARED((n, _LANES), jnp.float32),
    ]

  if with_deg:
    def body(feat, src, dst, zacc, zdeg, ones, acc_out, deg_out,
             sidx, didx, rows, acc_sh, sem, ones_v, deg_sh):
      _agg_body(feat, src, dst, zacc, acc_out, sidx, didx, rows, acc_sh,
                sem, zdeg=zdeg, ones=ones, deg_out=deg_out, ones_v=ones_v,
                deg_sh=deg_sh)
  else:
    def body(feat, src, dst, zacc, acc_out, sidx, didx, rows, acc_sh, sem):
      _agg_body(feat, src, dst, zacc, acc_out, sidx, didx, rows, acc_sh,
                sem)

  def _agg_body(feat, src, dst, zacc, acc_out, sidx, didx, rows, acc_sh,
                sem, zdeg=None, ones=None, deg_out=None, ones_v=None,
                deg_sh=None):
    c = lax.axis_index("c")
    s = lax.axis_index("s")
    wid = c * _NS + s
    rbase = s * rpt
    pltpu.sync_copy(zacc, acc_sh.at[pl.ds(rbase, rpt)])
    if with_deg:
      pltpu.sync_copy(zdeg, deg_sh.at[pl.ds(rbase, rpt)])
      pltpu.sync_copy(ones, ones_v)
    plsc.subcore_barrier()

    ebase = wid * epw

    def step(j, carry):
      base = ebase + j * _B
      pltpu.sync_copy(src.at[pl.ds(base, _B)], sidx)
      cp = pltpu.async_copy(feat.at[sidx], rows, sem)
      pltpu.sync_copy(dst.at[pl.ds(base, _B)], didx)
      cp.wait()
      pltpu.sync_copy(rows, acc_sh.at[didx], add=True)
      if with_deg:
        pltpu.sync_copy(ones_v, deg_sh.at[didx], add=True)
      return carry

    lax.fori_loop(0, nchunks, step, 0)
    plsc.subcore_barrier()
    pltpu.sync_copy(acc_sh.at[pl.ds(rbase, rpt)],
                    acc_out.at[c, pl.ds(rbase, rpt)])
    if with_deg:
      pltpu.sync_copy(deg_sh.at[pl.ds(rbase, rpt)],
                      deg_out.at[c, pl.ds(rbase, rpt)])

  return pl.kernel(body, out_type=out_type, mesh=mesh,
                   scratch_types=scratch,
                   compiler_params=pltpu.CompilerParams(
                       use_tc_tiling_on_sc=False))


def _tc_proj(x, w, b):
  n, din = x.shape
  dout = w.shape[0]
  bm = 1000

  def body(x_ref, w_ref, b_ref, o_ref):
    o_ref[...] = lax.dot_general(
        x_ref[...], w_ref[...], (((1,), (1,)), ((), ())),
        preferred_element_type=jnp.float32) + b_ref[...]

  return pl.pallas_call(
      body,
      grid=(n // bm,),
      in_specs=[pl.BlockSpec((bm, din), lambda i: (i, 0)),
                pl.BlockSpec((dout, din), lambda i: (0, 0)),
                pl.BlockSpec((1, dout), lambda i: (0, 0))],
      out_specs=pl.BlockSpec((bm, dout), lambda i: (i, 0)),
      out_shape=jax.ShapeDtypeStruct((n, dout), jnp.float32),
  )(x, w, b.reshape(1, dout))


def _tc_mid(acc, dega, xr, w1l, w2lp, w2rp, b2p):
  n = xr.shape[0]
  dh = xr.shape[1]
  dz = w2lp.shape[0]
  bm = 1000

  def body(acc_ref, deg_ref, xr_ref, wl_ref, w2l_ref, w2r_ref,
           b2_ref, y2_ref, hr_ref):
    a = acc_ref[0] + acc_ref[1]
    deg = deg_ref[0, :, 0:1] + deg_ref[1, :, 0:1]
    agg = a / jnp.maximum(deg, 1.0)
    pre = lax.dot_general(agg, wl_ref[...], (((1,), (1,)), ((), ())),
                          preferred_element_type=jnp.float32)
    h = jnp.maximum(pre + xr_ref[...], 0.0)
    y2_ref[...] = lax.dot_general(h, w2l_ref[...], (((1,), (1,)), ((), ())),
                                  preferred_element_type=jnp.float32)
    hr_ref[...] = lax.dot_general(h, w2r_ref[...], (((1,), (1,)), ((), ())),
                                  preferred_element_type=jnp.float32) + b2_ref[...]

  return pl.pallas_call(
      body,
      grid=(n // bm,),
      in_specs=[pl.BlockSpec((_NC, bm, dh), lambda i: (0, i, 0)),
                pl.BlockSpec((_NC, bm, _LANES), lambda i: (0, i, 0)),
                pl.BlockSpec((bm, dh), lambda i: (i, 0)),
                pl.BlockSpec((dh, dh), lambda i: (0, 0)),
                pl.BlockSpec((dz, dh), lambda i: (0, 0)),
                pl.BlockSpec((dz, dh), lambda i: (0, 0)),
                pl.BlockSpec((1, dz), lambda i: (0, 0))],
      out_specs=[pl.BlockSpec((bm, dz), lambda i: (i, 0)),
                 pl.BlockSpec((bm, dz), lambda i: (i, 0))],
      out_shape=[jax.ShapeDtypeStruct((n, dz), jnp.float32),
                 jax.ShapeDtypeStruct((n, dz), jnp.float32)],
  )(acc, dega, xr, w1l, w2lp, w2rp, b2p)


def _tc_final(acc2, dega, hr):
  n = hr.shape[0]
  dz = hr.shape[1]
  bm = 1000

  def body(acc_ref, deg_ref, hr_ref, o_ref):
    a = acc_ref[0] + acc_ref[1]
    deg = deg_ref[0, :, 0:1] + deg_ref[1, :, 0:1]
    o_ref[...] = a / jnp.maximum(deg, 1.0) + hr_ref[...]

  return pl.pallas_call(
      body,
      grid=(n // bm,),
      in_specs=[pl.BlockSpec((_NC, bm, dz), lambda i: (0, i, 0)),
                pl.BlockSpec((_NC, bm, _LANES), lambda i: (0, i, 0)),
                pl.BlockSpec((bm, dz), lambda i: (i, 0))],
      out_specs=pl.BlockSpec((bm, dz), lambda i: (i, 0)),
      out_shape=jax.ShapeDtypeStruct((n, dz), jnp.float32),
  )(acc2, dega, hr)


@jax.jit
def kernel(x, edge_index, W1_l, b1, W1_r, W2_l, b2, W2_r):
  n, din = x.shape
  e = edge_index.shape[1]
  dh = W1_l.shape[0]
  ncls = W2_l.shape[0]
  dz = ((ncls + _LANES - 1) // _LANES) * _LANES

  src = edge_index[0]
  dst = edge_index[1]
  npad = -(-n // (_NS * 8)) * (_NS * 8)
  rpt = npad // _NS
  zacc1 = jnp.zeros((rpt, din), jnp.float32)
  zdeg = jnp.zeros((rpt, _LANES), jnp.float32)
  ones = jnp.ones((_B, _LANES), jnp.float32)
  zacc2 = jnp.zeros((rpt, dz), jnp.float32)

  agg1 = _make_edge_agg(npad, e, din, with_deg=True)
  accx, dega = agg1(x, src, dst, zacc1, zdeg, ones)
  xr = _tc_proj(x, W1_r, b1)

  w2lp = jnp.pad(W2_l, ((0, dz - ncls), (0, 0)))
  w2rp = jnp.pad(W2_r, ((0, dz - ncls), (0, 0)))
  b2p = jnp.pad(b2, (0, dz - ncls)).reshape(1, dz)
  y2, hr = _tc_mid(accx, dega, xr, W1_l, w2lp, w2rp, b2p)

  agg2 = _make_edge_agg(npad, e, dz, with_deg=False)
  (acc2,) = agg2(y2, src, dst, zacc2)

  out = _tc_final(acc2, dega, hr)
  return out[:, :ncls]
